# baseline (speedup 1.0000x reference)
import jax
import jax.numpy as jnp
import math
from jax.experimental import pallas as pl

N = 10000
E = 320000


def _lin_pallas(pooled, lin_w, lin_b):
    def body(p_ref, w_ref, b_ref, o_ref):
        o_ref[...] = p_ref[...] @ w_ref[...].T + b_ref[...][None, :]

    return pl.pallas_call(
        body,
        out_shape=jax.ShapeDtypeStruct((1, lin_w.shape[0]), jnp.float32),
    )(pooled, lin_w, lin_b)


def _cayley_conv(x, h, alpha, Wr, Wc_re, Wc_im, l_row, l_col, deg, off_mask):
    R, KJ = 2, 3
    w_dia = deg - alpha
    tmp_left = 1.0 / (h * w_dia + 1j)
    off_row = l_row[:E]
    hw_off = (-h) * off_mask
    jac_off = -tmp_left[off_row] * hw_off
    jacobi = jnp.concatenate([jac_off, jnp.zeros((N,), jnp.complex64)])
    b_off = tmp_left[off_row] * hw_off
    b_dia = tmp_left * (h * w_dia - 1j)
    b_vals = jnp.concatenate([b_off, b_dia])
    out = x @ Wr.T
    y = x.astype(jnp.complex64)
    for j in range(R):
        b_j = jnp.zeros((N, y.shape[1]), jnp.complex64).at[l_row].add(b_vals[:, None] * y[l_col])
        y_k = b_j
        for _ in range(KJ):
            y_k = jnp.zeros((N, y_k.shape[1]), jnp.complex64).at[l_col].add(jacobi[:, None] * y_k[l_row]) + b_j
        y = y_k
        Wc = Wc_re[j] + 1j * Wc_im[j]
        out = out + 2.0 * jnp.real(y @ Wc.T)
    return out


def kernel(x, edge_index, h1, alpha1, Wr1, Wc1_re, Wc1_im, h2, alpha2, Wr2, Wc2_re, Wc2_im, pool_w, lin_w, lin_b):
    row, col = edge_index[0], edge_index[1]
    off_mask = (row != col).astype(jnp.float32)
    deg = jnp.zeros((N,), jnp.float32).at[row].add(off_mask)
    iota = jnp.arange(N, dtype=edge_index.dtype)
    l_row = jnp.concatenate([row, iota])
    l_col = jnp.concatenate([col, iota])
    x1 = jax.nn.relu(_cayley_conv(x, h1, alpha1, Wr1, Wc1_re, Wc1_im, l_row, l_col, deg, off_mask))
    x2 = jax.nn.relu(_cayley_conv(x1, h2, alpha2, Wr2, Wc2_re, Wc2_im, l_row, l_col, deg, off_mask))
    score = jnp.tanh((x2 @ pool_w) / jnp.linalg.norm(pool_w))
    k = int(math.ceil(0.9 * N))
    vals, perm = jax.lax.top_k(score, k)
    xp = x2[perm] * vals[:, None]
    pooled = jnp.mean(xp, axis=0, keepdims=True)
    return _lin_pallas(pooled, lin_w, lin_b)


# speedup vs baseline: 31.8619x; 31.8619x over previous
import functools

import jax
import jax.numpy as jnp
from jax import lax
from jax.experimental import pallas as pl
from jax.experimental.pallas import tpu as pltpu
from jax.experimental.pallas import tpu_sc as plsc

N = 10000
E = 320000
H = 128
OUT = 64
R = 2
KJ = 3
K_POOL = 9000

NC = 2
NS = 16
L = 16

NPAD = 10240
RPT = NPAD // NS
CHUNK = 1024
BLK = 128


def _mesh():
    return plsc.VectorSubcoreMesh(
        core_axis_name="c", subcore_axis_name="s", num_cores=NC,
        num_subcores=NS)


NCHD = 20


@functools.partial(
    pl.kernel,
    out_type=[jax.ShapeDtypeStruct((NPAD,), jnp.float32)] * 4,
    mesh=_mesh(),
    compiler_params=pltpu.CompilerParams(needs_layout_passes=False),
    scratch_types=[
        pltpu.VMEM_SHARED((NPAD,), jnp.float32),
        pltpu.VMEM((8, BLK), jnp.int32),
        pltpu.VMEM((8, BLK), jnp.int32),
        pltpu.VMEM((8, BLK), jnp.float32),
        pltpu.VMEM((L,), jnp.float32),
        pltpu.VMEM((L,), jnp.float32),
        pltpu.VMEM((RPT,), jnp.float32),
        pltpu.VMEM((RPT,), jnp.float32),
        pltpu.VMEM((RPT,), jnp.float32),
        pltpu.VMEM((RPT,), jnp.float32),
        pltpu.VMEM((RPT,), jnp.float32),
    ],
)
def _prep_kernel(row_h, col_h, h_h, a_h, zer_h,
                 gre_o, gim_o, bdre_o, bdim_o,
                 deg_sh, row_v, col_v, val_v, hv, av,
                 dbuf, gre_b, gim_b, bre_b, bim_b):
    c = lax.axis_index("c")
    s = lax.axis_index("s")

    @pl.when(c == 0)
    def _():
        @pl.when(s == 0)
        def _():
            pltpu.sync_copy(zer_h, deg_sh)

        plsc.subcore_barrier()

        def chunk(ch, carry):
            slot = s * NCHD + ch
            pltpu.sync_copy(row_h.at[slot], row_v)
            pltpu.sync_copy(col_h.at[slot], col_v)
            for j in range(8):
                for q in range(8):
                    sl = pl.ds(q * L, L)
                    r16 = row_v[j, sl]
                    c16 = col_v[j, sl]
                    val_v[j, sl] = jnp.where(
                        r16 == c16, jnp.float32(0.0), jnp.float32(1.0))
                pltpu.sync_copy(val_v.at[j], deg_sh.at[row_v.at[j]],
                                add=True)
            return carry

        lax.fori_loop(0, NCHD, chunk, 0)
        plsc.subcore_barrier()

        pltpu.sync_copy(h_h, hv)
        pltpu.sync_copy(a_h, av)
        r0 = s * RPT
        pltpu.sync_copy(deg_sh.at[pl.ds(r0, RPT)], dbuf)
        hh = hv[...]
        aa = av[...]
        for i in range(RPT // L):
            sl = pl.ds(i * L, L)
            wd = dbuf[sl] - aa
            cc = hh * wd
            den = cc * cc + 1.0
            tre = cc / den
            tim = -1.0 / den
            gre_b[sl] = hh * tre
            gim_b[sl] = hh * tim
            bre_b[sl] = tre * cc + tim
            bim_b[sl] = tim * cc - tre
        pltpu.sync_copy(gre_b, gre_o.at[pl.ds(r0, RPT)])
        pltpu.sync_copy(gim_b, gim_o.at[pl.ds(r0, RPT)])
        pltpu.sync_copy(bre_b, bdre_o.at[pl.ds(r0, RPT)])
        pltpu.sync_copy(bim_b, bdim_o.at[pl.ds(r0, RPT)])


NCHW = 10
NSLOTW = NC * NS * NCHW


@functools.partial(
    pl.kernel,
    out_type=[jax.ShapeDtypeStruct((NSLOTW, 8, BLK), jnp.float32)] * 2,
    mesh=_mesh(),
    compiler_params=pltpu.CompilerParams(needs_layout_passes=False),
    scratch_types=[
        pltpu.VMEM((NPAD,), jnp.float32),
        pltpu.VMEM((NPAD,), jnp.float32),
        pltpu.VMEM((8, BLK), jnp.int32),
        pltpu.VMEM((8, BLK), jnp.int32),
        pltpu.VMEM((8, BLK), jnp.float32),
        pltpu.VMEM((8, BLK), jnp.float32),
    ],
)
def _edgew_kernel(row_h, col_h, gre_h, gim_h,
                  jre_o, jim_o,
                  gre_v, gim_v, row_v, col_v, jre_b, jim_b):
    c = lax.axis_index("c")
    s = lax.axis_index("s")
    wid = s * NC + c
    pltpu.sync_copy(gre_h, gre_v)
    pltpu.sync_copy(gim_h, gim_v)

    def chunk(ch, carry):
        slot = wid * NCHW + ch
        pltpu.sync_copy(row_h.at[slot], row_v)
        pltpu.sync_copy(col_h.at[slot], col_v)
        for j in range(8):
            for q in range(8):
                sl = pl.ds(q * L, L)
                r16 = row_v[j, sl]
                c16 = col_v[j, sl]
                gr = plsc.load_gather(gre_v, [r16])
                gi = plsc.load_gather(gim_v, [r16])
                neq = r16 != c16
                jre_b[j, sl] = jnp.where(neq, gr, jnp.float32(0.0))
                jim_b[j, sl] = jnp.where(neq, gi, jnp.float32(0.0))
        pltpu.sync_copy(jre_b, jre_o.at[slot])
        pltpu.sync_copy(jim_b, jim_o.at[slot])
        return carry

    lax.fori_loop(0, NCHW, chunk, 0)


_SPMM_CACHE = {}


def _spmm_kernel(nch):
    if nch in _SPMM_CACHE:
        return _SPMM_CACHE[nch]

    nslot = NS * nch

    @functools.partial(
        pl.kernel,
        out_type=[jax.ShapeDtypeStruct((NPAD, H), jnp.float32)] * 2,
        mesh=_mesh(),
        compiler_params=pltpu.CompilerParams(needs_layout_passes=False),
        scratch_types=[
            pltpu.VMEM_SHARED((NPAD, H), jnp.float32),
            pltpu.VMEM((8, BLK), jnp.int32),
            pltpu.VMEM((8, BLK), jnp.int32),
            pltpu.VMEM((8, BLK), jnp.float32),
            pltpu.VMEM((8, BLK), jnp.float32),
            pltpu.VMEM((BLK, H), jnp.float32),
            pltpu.VMEM((BLK, H), jnp.float32),
        ],
    )
    def kern(y0_h, y1_h, init0_h, init1_h, src_h, dst_h, wre_h, wim_h,
             out0_h, out1_h,
             acc, src_v, dst_v, wre_v, wim_v, gbuf, pbuf):
        c = lax.axis_index("c")
        s = lax.axis_index("s")
        r0 = s * RPT

        def run(y_h, init_h, out_h):
            pltpu.sync_copy(init_h.at[pl.ds(r0, RPT), :],
                            acc.at[pl.ds(r0, RPT), :])
            plsc.subcore_barrier()

            def chunk(ch, carry):
                slot = s * nch + ch
                pltpu.sync_copy(src_h.at[slot], src_v)
                pltpu.sync_copy(dst_h.at[slot], dst_v)
                pltpu.sync_copy(wre_h.at[slot], wre_v)
                pltpu.sync_copy(wim_h.at[slot], wim_v)
                for j in range(8):
                    pltpu.sync_copy(y_h.at[src_v.at[j]], gbuf)

                    def edge(e, cr):
                        jv = jnp.full((L,), j, jnp.int32)
                        ev = jnp.full((L,), e, jnp.int32)
                        wr = plsc.load_gather(wre_v, [jv, ev])
                        wi = plsc.load_gather(wim_v, [jv, ev])
                        for q in range(4):
                            slr = pl.ds(q * L, L)
                            sli = pl.ds(64 + q * L, L)
                            a = gbuf[e, slr]
                            b = gbuf[e, sli]
                            pbuf[e, slr] = wr * a - wi * b
                            pbuf[e, sli] = wr * b + wi * a
                        return cr

                    lax.fori_loop(0, BLK, edge, 0)
                    pltpu.sync_copy(pbuf, acc.at[dst_v.at[j]], add=True)
                return carry

            lax.fori_loop(0, nch, chunk, 0)
            plsc.subcore_barrier()
            pltpu.sync_copy(acc.at[pl.ds(r0, RPT), :],
                            out_h.at[pl.ds(r0, RPT), :])

        @pl.when(c == 0)
        def _():
            run(y0_h, init0_h, out0_h)

        @pl.when(c == 1)
        def _():
            run(y1_h, init1_h, out1_h)

    _SPMM_CACHE[nch] = kern
    return kern


DBLK = 400


def _dense_body(x_ref, p00, p01, p10, p11, wr_ref, wcre_ref, wcim_ref,
                pw_ref, o_ref, z_ref):
    acc = jnp.dot(x_ref[...], wr_ref[...].T,
                  preferred_element_type=jnp.float32)
    for r, (pa, pb) in enumerate(((p00, p01), (p10, p11))):
        yre = jnp.concatenate([pa[...][:, :64], pb[...][:, :64]], axis=1)
        yim = jnp.concatenate([pa[...][:, 64:], pb[...][:, 64:]], axis=1)
        acc = acc + 2.0 * (
            jnp.dot(yre, wcre_ref[r].T, preferred_element_type=jnp.float32)
            - jnp.dot(yim, wcim_ref[r].T,
                      preferred_element_type=jnp.float32))
    xo = jnp.maximum(acc, 0.0)
    o_ref[...] = xo
    z_ref[...] = jnp.dot(xo, pw_ref[...].T,
                         preferred_element_type=jnp.float32)


def _dense(x, p00, p01, p10, p11, Wr, Wcre, Wcim, pw):
    nblk = N // DBLK
    row_spec = pl.BlockSpec((DBLK, H), lambda i: (i, 0))
    full = lambda shape: pl.BlockSpec(shape, lambda i: (0,) * len(shape))
    return pl.pallas_call(
        _dense_body,
        grid=(nblk,),
        in_specs=[row_spec, row_spec, row_spec, row_spec, row_spec,
                  full((H, H)), full((R, H, H)), full((R, H, H)),
                  full((1, H))],
        out_specs=[pl.BlockSpec((DBLK, H), lambda i: (i, 0)),
                   pl.BlockSpec((DBLK, 1), lambda i: (i, 0))],
        out_shape=[jax.ShapeDtypeStruct((N, H), jnp.float32),
                   jax.ShapeDtypeStruct((N, 1), jnp.float32)],
    )(x, p00, p01, p10, p11, Wr, Wcre, Wcim, pw)


ZR, ZC = 80, 125


def _thresh_body(z_ref, pw_ref, o_ref):
    pw = pw_ref[...]
    nrm = jnp.sqrt(jnp.sum(pw * pw))
    s = jnp.tanh(z_ref[...] / nrm)
    b = lax.bitcast_convert_type(s, jnp.uint32)
    ukey = jnp.where((b >> jnp.uint32(31)) != 0, ~b,
                     b | jnp.uint32(0x80000000))

    def step(i, t):
        cand = t | jnp.left_shift(jnp.uint32(1),
                                  (31 - i).astype(jnp.uint32))
        cnt = jnp.sum((ukey >= cand).astype(jnp.int32))
        return jnp.where(cnt >= K_POOL, cand, t)

    tau = lax.fori_loop(0, 32, step, jnp.uint32(0))
    gt = ukey > tau
    m = K_POOL - jnp.sum(gt.astype(jnp.int32))
    tie = ukey == tau
    ridx = (lax.broadcasted_iota(jnp.int32, (ZR, ZC), 0) * ZC
            + lax.broadcasted_iota(jnp.int32, (ZR, ZC), 1))

    def step2(i, t):
        cand = t | jnp.left_shift(jnp.int32(1), 14 - i)
        cnt = jnp.sum((tie & (ridx < cand)).astype(jnp.int32))
        return jnp.where(cnt <= m, cand, t)

    tcut = lax.fori_loop(0, 15, step2, jnp.int32(0))
    sel = gt | (tie & (ridx < tcut))
    o_ref[...] = jnp.where(sel, s, jnp.float32(0.0))


def _thresh(z80, pw):
    return pl.pallas_call(
        _thresh_body,
        out_shape=jax.ShapeDtypeStruct((ZR, ZC), jnp.float32),
    )(z80, pw)


def _final_body(w_ref, x2_ref, lw_ref, lb_ref, o_ref):
    pooled = jnp.dot(w_ref[...], x2_ref[...],
                     preferred_element_type=jnp.float32) / jnp.float32(
                         K_POOL)
    o_ref[...] = jnp.dot(pooled, lw_ref[...].T,
                         preferred_element_type=jnp.float32) + lb_ref[...]


def _final(w_row, x2, lin_w, lin_b):
    return pl.pallas_call(
        _final_body,
        out_shape=jax.ShapeDtypeStruct((1, OUT), jnp.float32),
    )(w_row, x2, lin_w, lin_b.reshape(1, OUT))


def _edge_chunks(arr, per_tile, ntiles, fill):
    tot = per_tile * ntiles
    pad = jnp.full((tot - arr.shape[0],), fill, arr.dtype)
    return jnp.concatenate([arr, pad]).reshape(-1, 8, BLK)


def _planes_from_real(x):
    z64 = jnp.zeros((N, 64), jnp.float32)
    rows = NPAD - N
    p0 = jnp.pad(jnp.concatenate([x[:, :64], z64], axis=1),
                 ((0, rows), (0, 0)))
    p1 = jnp.pad(jnp.concatenate([x[:, 64:], z64], axis=1),
                 ((0, rows), (0, 0)))
    return p0, p1


def kernel(x, edge_index, h1, alpha1, Wr1, Wc1_re, Wc1_im,
           h2, alpha2, Wr2, Wc2_re, Wc2_im, pool_w, lin_w, lin_b):
    row = edge_index[0]
    col = edge_index[1]
    iota = jnp.arange(NPAD, dtype=jnp.int32)

    row_d = _edge_chunks(row, NCHD * CHUNK, NS, 0)
    col_d = _edge_chunks(col, NCHD * CHUNK, NS, 0)
    row_w = _edge_chunks(row, NCHW * CHUNK, NC * NS, 0)
    col_w = _edge_chunks(col, NCHW * CHUNK, NC * NS, 0)

    nchb = -(-(E + NPAD) // (NS * CHUNK))
    src_b = _edge_chunks(jnp.concatenate([col, iota]), nchb * CHUNK, NS, N)
    dst_b = _edge_chunks(jnp.concatenate([row, iota]), nchb * CHUNK, NS, N)
    nchj = -(-E // (NS * CHUNK))
    src_j = _edge_chunks(row, nchj * CHUNK, NS, N)
    dst_j = _edge_chunks(col, nchj * CHUNK, NS, N)

    zero_pad = jnp.zeros((NPAD,), jnp.float32)
    zero_plane = jnp.zeros((NPAD, H), jnp.float32)

    x_cur = x
    planes = _planes_from_real(x)
    z_score = None

    for (h, alpha, Wr, Wcre, Wcim) in (
            (h1, alpha1, Wr1, Wc1_re, Wc1_im),
            (h2, alpha2, Wr2, Wc2_re, Wc2_im)):
        h16 = jnp.full((L,), h, jnp.float32)
        a16 = jnp.full((L,), alpha, jnp.float32)
        gre, gim, bdre, bdim = _prep_kernel(
            row_d, col_d, h16, a16, zero_pad)
        jre, jim = _edgew_kernel(row_w, col_w, gre, gim)
        jre = jre.reshape(-1)[:E]
        jim = jim.reshape(-1)[:E]

        wre_b = _edge_chunks(jnp.concatenate([-jre, bdre]),
                             nchb * CHUNK, NS, 0.0)
        wim_b = _edge_chunks(jnp.concatenate([-jim, bdim]),
                             nchb * CHUNK, NS, 0.0)
        wre_j = _edge_chunks(jre, nchj * CHUNK, NS, 0.0)
        wim_j = _edge_chunks(jim, nchj * CHUNK, NS, 0.0)

        spmm_b = _spmm_kernel(nchb)
        spmm_j = _spmm_kernel(nchj)

        y = planes
        ys = []
        for _ in range(R):
            bj = spmm_b(y[0], y[1], zero_plane, zero_plane,
                        src_b, dst_b, wre_b, wim_b)
            yk = bj
            for _ in range(KJ):
                yk = spmm_j(yk[0], yk[1], bj[0], bj[1],
                            src_j, dst_j, wre_j, wim_j)
            ys.append(yk)
            y = yk

        x_cur, z_score = _dense(x_cur, ys[0][0], ys[0][1], ys[1][0],
                                ys[1][1], Wr, Wcre, Wcim,
                                pool_w.reshape(1, H))
        planes = _planes_from_real(x_cur)

    w80 = _thresh(z_score.reshape(ZR, ZC), pool_w.reshape(1, H))
    return _final(w80.reshape(1, N), x_cur, lin_w, lin_b)
